# baseline (speedup 1.0000x reference)
import jax
import jax.numpy as jnp
from jax import lax
from jax.experimental import pallas as pl
from jax.experimental.pallas import tpu as pltpu

B = 32
P = 8732
C = 21
M = 20
ROWS = 69
LANE = 128
PP = ROWS * LANE
THRESHOLD = 0.5
NEG_POS_RATIO = 3
ALPHA = 1.0
BIG = 2 ** 30


def _match_body(locs_ref, priors_ref, bxy_ref, bcxcy_ref, labels_ref,
                lab_ref, npos_ref, locsum_ref):
    r_iota = lax.broadcasted_iota(jnp.int32, (ROWS, LANE), 0)
    l_iota = lax.broadcasted_iota(jnp.int32, (ROWS, LANE), 1)
    p_idx = r_iota * LANE + l_iota

    px0 = priors_ref[0]
    py0 = priors_ref[1]
    px1 = priors_ref[2]
    py1 = priors_ref[3]
    pcx = priors_ref[4]
    pcy = priors_ref[5]
    pw = priors_ref[6]
    ph = priors_ref[7]
    parea = (px1 - px0) * (py1 - py0)

    rows = []
    for m in range(M):
        bx0 = bxy_ref[0, m, 0]
        by0 = bxy_ref[0, m, 1]
        bx1 = bxy_ref[0, m, 2]
        by1 = bxy_ref[0, m, 3]
        iw = jnp.clip(jnp.minimum(bx1, px1) - jnp.maximum(bx0, px0), 0.0)
        ih = jnp.clip(jnp.minimum(by1, py1) - jnp.maximum(by0, py0), 0.0)
        inter = iw * ih
        barea = (bx1 - bx0) * (by1 - by0)
        rows.append(inter / (barea + parea - inter + 1e-8))
    iou3 = jnp.stack(rows, axis=0)
    m_iota = lax.broadcasted_iota(jnp.int32, (M, ROWS, LANE), 0)
    p_idx3 = p_idx[None]

    best_val = jnp.max(iou3, axis=0)
    best_obj = jnp.min(jnp.where(iou3 == best_val[None], m_iota, BIG),
                       axis=0)
    row_max = jnp.max(iou3, axis=(1, 2), keepdims=True)
    prior_m = jnp.min(jnp.where(iou3 == row_max, p_idx3, BIG),
                      axis=(1, 2), keepdims=True)

    eqp = p_idx3 == prior_m
    forced = jnp.max(jnp.where(eqp, m_iota, -1), axis=0)
    is_forced = forced >= 0
    best_obj = jnp.where(is_forced, forced, best_obj)
    best_val = jnp.where(is_forced, 1.0, best_val)

    lab = jnp.zeros((ROWS, LANE), jnp.int32)
    bcx = jnp.zeros((ROWS, LANE), jnp.float32)
    bcy = jnp.zeros((ROWS, LANE), jnp.float32)
    bw = jnp.zeros((ROWS, LANE), jnp.float32)
    bh = jnp.zeros((ROWS, LANE), jnp.float32)
    for m in range(M):
        is_m = best_obj == m
        lab = jnp.where(is_m, labels_ref[0, 0, m], lab)
        bcx = jnp.where(is_m, bcxcy_ref[0, m, 0], bcx)
        bcy = jnp.where(is_m, bcxcy_ref[0, m, 1], bcy)
        bw = jnp.where(is_m, bcxcy_ref[0, m, 2], bw)
        bh = jnp.where(is_m, bcxcy_ref[0, m, 3], bh)
    lab = jnp.where(best_val < THRESHOLD, 0, lab)
    positive = lab != 0
    n_pos = jnp.sum(positive.astype(jnp.float32), keepdims=True)

    g0 = (bcx - pcx) / (pw / 10.0)
    g1 = (bcy - pcy) / (ph / 10.0)
    g2 = jnp.log(jnp.maximum(bw / pw, 1e-8)) * 5.0
    g3 = jnp.log(jnp.maximum(bh / ph, 1e-8)) * 5.0
    sl1_sum = jnp.zeros((ROWS, LANE), jnp.float32)
    for j, g in enumerate((g0, g1, g2, g3)):
        d = locs_ref[0, j] - g
        ad = jnp.abs(d)
        sl1_sum = sl1_sum + jnp.where(ad < 1.0, 0.5 * ad * ad, ad - 0.5)
    loc_sum = jnp.sum(jnp.where(positive, sl1_sum, 0.0), keepdims=True)

    lab_ref[0] = lab
    npos_ref[0] = n_pos
    locsum_ref[0] = loc_sum


def _conf_body(scores_ref, lab_in_ref, npos_in_ref, locsum_in_ref,
               out_ref, ce_ref, confpos_ref):
    i = pl.program_id(0)

    r_iota = lax.broadcasted_iota(jnp.int32, (ROWS, LANE), 0)
    l_iota = lax.broadcasted_iota(jnp.int32, (ROWS, LANE), 1)
    valid = (r_iota * LANE + l_iota) < P

    lab = lab_in_ref[0]
    positive = lab != 0

    maxv = scores_ref[0, 0]
    for c in range(1, C):
        maxv = jnp.maximum(maxv, scores_ref[0, c])
    sumexp = jnp.zeros((ROWS, LANE), jnp.float32)
    s_lab = jnp.zeros((ROWS, LANE), jnp.float32)
    for c in range(C):
        s = scores_ref[0, c]
        sumexp = sumexp + jnp.exp(s - maxv)
        s_lab = jnp.where(lab == c, s, s_lab)
    ce = (maxv - s_lab) + jnp.log(sumexp)
    conf_pos = jnp.sum(jnp.where(positive, ce, 0.0), keepdims=True)
    ce_neg = jnp.where(positive | jnp.logical_not(valid), 0.0, ce)

    ce_ref[i] = ce_neg
    confpos_ref[pl.ds(i, 1), :] = conf_pos

    @pl.when(i == B - 1)
    def finalize():
        x = ce_ref[...]
        xb = lax.bitcast_convert_type(x, jnp.int32)
        k_f = npos_in_ref[...] * float(NEG_POS_RATIO)

        def search_body(t, u):
            v = u | (jnp.int32(1) << (jnp.int32(30) - t))
            cnt = jnp.sum(jnp.where(xb >= v, 1.0, 0.0),
                          axis=(1, 2), keepdims=True)
            return jnp.where(cnt >= k_f, v, u)

        u = lax.fori_loop(0, 31, search_body,
                          jnp.zeros((B, 1, 1), jnp.int32))
        gt = xb > u
        cnt_gt = jnp.sum(jnp.where(gt, 1.0, 0.0), axis=(1, 2), keepdims=True)
        sum_gt = jnp.sum(jnp.where(gt, x, 0.0), axis=(1, 2), keepdims=True)
        t_star = jnp.max(jnp.where(xb <= u, x, 0.0),
                         axis=(1, 2), keepdims=True)
        topk = sum_gt + (k_f - cnt_gt) * t_star

        conf_total = (jnp.sum(confpos_ref[...], axis=0, keepdims=True)
                      + jnp.sum(topk, axis=0))
        loc_total = jnp.sum(locsum_in_ref[...], axis=0)
        n_total = jnp.maximum(jnp.sum(npos_in_ref[...], axis=0), 1.0)
        out_ref[:, :] = conf_total / n_total + ALPHA * loc_total / (n_total * 4.0)


def kernel(predicted_locs, predicted_scores, boxes, labels, priors_cxcy):
    pad = PP - P
    scores_t = jnp.pad(jnp.transpose(predicted_scores, (0, 2, 1)),
                       ((0, 0), (0, 0), (0, pad))).reshape(B, C, ROWS, LANE)
    locs_t = jnp.pad(jnp.transpose(predicted_locs, (0, 2, 1)),
                     ((0, 0), (0, 0), (0, pad))).reshape(B, 4, ROWS, LANE)
    priors_xy = jnp.concatenate(
        [priors_cxcy[..., :2] - priors_cxcy[..., 2:] / 2.0,
         priors_cxcy[..., :2] + priors_cxcy[..., 2:] / 2.0], axis=-1)
    pri = jnp.pad(jnp.transpose(
        jnp.concatenate([priors_xy, priors_cxcy], axis=-1), (1, 0)),
        ((0, 0), (0, pad))).reshape(8, ROWS, LANE)
    bcxcy = jnp.concatenate([(boxes[..., :2] + boxes[..., 2:]) / 2.0,
                             boxes[..., 2:] - boxes[..., :2]], axis=-1)
    labels32 = labels.astype(jnp.int32).reshape(B, 1, M)

    def img_map(i):
        return (i, 0, 0, 0)

    def img_map3(i):
        return (i, 0, 0)

    lab_arr, npos_arr, locsum_arr = pl.pallas_call(
        _match_body,
        grid=(B,),
        in_specs=[
            pl.BlockSpec((1, 4, ROWS, LANE), img_map),
            pl.BlockSpec((8, ROWS, LANE), lambda i: (0, 0, 0)),
            pl.BlockSpec((1, M, 4), img_map3, memory_space=pltpu.SMEM),
            pl.BlockSpec((1, M, 4), img_map3, memory_space=pltpu.SMEM),
            pl.BlockSpec((1, 1, M), img_map3, memory_space=pltpu.SMEM),
        ],
        out_specs=[
            pl.BlockSpec((1, ROWS, LANE), img_map3),
            pl.BlockSpec((1, 1, 1), img_map3),
            pl.BlockSpec((1, 1, 1), img_map3),
        ],
        out_shape=[
            jax.ShapeDtypeStruct((B, ROWS, LANE), jnp.int32),
            jax.ShapeDtypeStruct((B, 1, 1), jnp.float32),
            jax.ShapeDtypeStruct((B, 1, 1), jnp.float32),
        ],
    )(locs_t, pri, boxes, bcxcy, labels32)

    out = pl.pallas_call(
        _conf_body,
        grid=(B,),
        in_specs=[
            pl.BlockSpec((1, C, ROWS, LANE), img_map),
            pl.BlockSpec((1, ROWS, LANE), img_map3),
            pl.BlockSpec((B, 1, 1), lambda i: (0, 0, 0)),
            pl.BlockSpec((B, 1, 1), lambda i: (0, 0, 0)),
        ],
        out_specs=pl.BlockSpec((1, 1), lambda i: (0, 0)),
        out_shape=jax.ShapeDtypeStruct((1, 1), jnp.float32),
        scratch_shapes=[
            pltpu.VMEM((B, ROWS, LANE), jnp.float32),
            pltpu.VMEM((B, 1), jnp.float32),
        ],
    )(scores_t, lab_arr, npos_arr, locsum_arr)
    return out[0, 0]


# speedup vs baseline: 1.0904x; 1.0904x over previous
import jax
import jax.numpy as jnp
from jax import lax
from jax.experimental import pallas as pl
from jax.experimental.pallas import tpu as pltpu

B = 32
P = 8732
C = 21
M = 20
ROWS = 69
LANE = 128
PP = ROWS * LANE
FULL = P // LANE
TAIL = P - FULL * LANE
THRESHOLD = 0.5
NEG_POS_RATIO = 3
ALPHA = 1.0
BIG = 2 ** 30


def _body(scores_ref, locs_ref, priors_ref, bxy_ref, bcxcy_ref, labels_ref,
          out_ref, ce_ref, npos_ref, confpos_ref, locsum_ref):
    i = pl.program_id(0)

    r_iota = lax.broadcasted_iota(jnp.int32, (ROWS, LANE), 0)
    l_iota = lax.broadcasted_iota(jnp.int32, (ROWS, LANE), 1)
    p_idx = r_iota * LANE + l_iota
    valid = p_idx < P

    px0 = priors_ref[0]
    py0 = priors_ref[1]
    px1 = priors_ref[2]
    py1 = priors_ref[3]
    pcx = priors_ref[4]
    pcy = priors_ref[5]
    pw = priors_ref[6]
    ph = priors_ref[7]
    parea = (px1 - px0) * (py1 - py0)

    rows = []
    for m in range(M):
        bx0 = bxy_ref[0, m, 0]
        by0 = bxy_ref[0, m, 1]
        bx1 = bxy_ref[0, m, 2]
        by1 = bxy_ref[0, m, 3]
        iw = jnp.clip(jnp.minimum(bx1, px1) - jnp.maximum(bx0, px0), 0.0)
        ih = jnp.clip(jnp.minimum(by1, py1) - jnp.maximum(by0, py0), 0.0)
        inter = iw * ih
        barea = (bx1 - bx0) * (by1 - by0)
        rows.append(inter / (barea + parea - inter + 1e-8))
    iou3 = jnp.stack(rows, axis=0)
    m_iota = lax.broadcasted_iota(jnp.int32, (M, ROWS, LANE), 0)
    p_idx3 = p_idx[None]

    best_val = jnp.max(iou3, axis=0)
    best_obj = jnp.min(jnp.where(iou3 == best_val[None], m_iota, BIG),
                       axis=0)
    row_max = jnp.max(iou3, axis=(1, 2), keepdims=True)
    prior_m = jnp.min(jnp.where(iou3 == row_max, p_idx3, BIG),
                      axis=(1, 2), keepdims=True)

    eqp = p_idx3 == prior_m
    forced = jnp.max(jnp.where(eqp, m_iota, -1), axis=0)
    is_forced = forced >= 0
    best_obj = jnp.where(is_forced, forced, best_obj)
    best_val = jnp.where(is_forced, 1.0, best_val)

    lab = jnp.zeros((ROWS, LANE), jnp.int32)
    bcx = jnp.zeros((ROWS, LANE), jnp.float32)
    bcy = jnp.zeros((ROWS, LANE), jnp.float32)
    bw = jnp.zeros((ROWS, LANE), jnp.float32)
    bh = jnp.zeros((ROWS, LANE), jnp.float32)
    for m in range(M):
        is_m = best_obj == m
        lab = jnp.where(is_m, labels_ref[0, 0, m], lab)
        bcx = jnp.where(is_m, bcxcy_ref[0, m, 0], bcx)
        bcy = jnp.where(is_m, bcxcy_ref[0, m, 1], bcy)
        bw = jnp.where(is_m, bcxcy_ref[0, m, 2], bw)
        bh = jnp.where(is_m, bcxcy_ref[0, m, 3], bh)
    lab = jnp.where(best_val < THRESHOLD, 0, lab)
    positive = lab != 0
    n_pos = jnp.sum(positive.astype(jnp.float32), keepdims=True)

    g0 = (bcx - pcx) / (pw / 10.0)
    g1 = (bcy - pcy) / (ph / 10.0)
    g2 = jnp.log(jnp.maximum(bw / pw, 1e-8)) * 5.0
    g3 = jnp.log(jnp.maximum(bh / ph, 1e-8)) * 5.0
    sl1_sum = jnp.zeros((ROWS, LANE), jnp.float32)
    for j, g in enumerate((g0, g1, g2, g3)):
        d = locs_ref[0, j] - g
        ad = jnp.abs(d)
        sl1_sum = sl1_sum + jnp.where(ad < 1.0, 0.5 * ad * ad, ad - 0.5)
    loc_sum = jnp.sum(jnp.where(positive, sl1_sum, 0.0), keepdims=True)

    maxv = scores_ref[0, 0]
    for c in range(1, C):
        maxv = jnp.maximum(maxv, scores_ref[0, c])
    sumexp = jnp.zeros((ROWS, LANE), jnp.float32)
    s_lab = jnp.zeros((ROWS, LANE), jnp.float32)
    for c in range(C):
        s = scores_ref[0, c]
        sumexp = sumexp + jnp.exp(s - maxv)
        s_lab = jnp.where(lab == c, s, s_lab)
    ce = (maxv - s_lab) + jnp.log(sumexp)
    conf_pos = jnp.sum(jnp.where(positive, ce, 0.0), keepdims=True)
    ce_neg = jnp.where(positive | jnp.logical_not(valid), 0.0, ce)

    ce_ref[i] = ce_neg
    npos_ref[pl.ds(i, 1), :] = n_pos
    confpos_ref[pl.ds(i, 1), :] = conf_pos
    locsum_ref[pl.ds(i, 1), :] = loc_sum

    @pl.when(i == B - 1)
    def finalize():
        x = ce_ref[...]
        xb = lax.bitcast_convert_type(x, jnp.int32)
        k_f = npos_ref[...].reshape(B, 1, 1) * float(NEG_POS_RATIO)

        def search_body(t, u):
            v = u | (jnp.int32(1) << (jnp.int32(30) - t))
            cnt = jnp.sum(jnp.where(xb >= v, 1.0, 0.0),
                          axis=(1, 2), keepdims=True)
            return jnp.where(cnt >= k_f, v, u)

        u = lax.fori_loop(0, 31, search_body,
                          jnp.zeros((B, 1, 1), jnp.int32))
        gt = xb > u
        cnt_gt = jnp.sum(jnp.where(gt, 1.0, 0.0), axis=(1, 2), keepdims=True)
        sum_gt = jnp.sum(jnp.where(gt, x, 0.0), axis=(1, 2), keepdims=True)
        t_star = jnp.max(jnp.where(xb <= u, x, 0.0),
                         axis=(1, 2), keepdims=True)
        topk = sum_gt + (k_f - cnt_gt) * t_star

        conf_total = (jnp.sum(confpos_ref[...], axis=0, keepdims=True)
                      + jnp.sum(topk, axis=0))
        loc_total = jnp.sum(locsum_ref[...], axis=0, keepdims=True)
        n_total = jnp.maximum(jnp.sum(npos_ref[...], axis=0, keepdims=True),
                              1.0)
        out_ref[:, :] = conf_total / n_total + ALPHA * loc_total / (n_total * 4.0)


def kernel(predicted_locs, predicted_scores, boxes, labels, priors_cxcy):
    pad = PP - P
    scores_t = jnp.pad(jnp.transpose(predicted_scores, (0, 2, 1)),
                       ((0, 0), (0, 0), (0, pad))).reshape(B, C, ROWS, LANE)
    locs_t = jnp.pad(jnp.transpose(predicted_locs, (0, 2, 1)),
                     ((0, 0), (0, 0), (0, pad))).reshape(B, 4, ROWS, LANE)
    priors_xy = jnp.concatenate(
        [priors_cxcy[..., :2] - priors_cxcy[..., 2:] / 2.0,
         priors_cxcy[..., :2] + priors_cxcy[..., 2:] / 2.0], axis=-1)
    pri = jnp.pad(jnp.transpose(
        jnp.concatenate([priors_xy, priors_cxcy], axis=-1), (1, 0)),
        ((0, 0), (0, pad))).reshape(8, ROWS, LANE)
    bcxcy = jnp.concatenate([(boxes[..., :2] + boxes[..., 2:]) / 2.0,
                             boxes[..., 2:] - boxes[..., :2]], axis=-1)
    labels32 = labels.astype(jnp.int32).reshape(B, 1, M)

    grid = (B,)

    def img_map(i):
        return (i, 0, 0, 0)

    def img_map3(i):
        return (i, 0, 0)

    out = pl.pallas_call(
        _body,
        grid=grid,
        in_specs=[
            pl.BlockSpec((1, C, ROWS, LANE), img_map),
            pl.BlockSpec((1, 4, ROWS, LANE), img_map),
            pl.BlockSpec((8, ROWS, LANE), lambda i: (0, 0, 0)),
            pl.BlockSpec((1, M, 4), img_map3, memory_space=pltpu.SMEM),
            pl.BlockSpec((1, M, 4), img_map3, memory_space=pltpu.SMEM),
            pl.BlockSpec((1, 1, M), img_map3, memory_space=pltpu.SMEM),
        ],
        out_specs=pl.BlockSpec((1, 1), lambda i: (0, 0)),
        out_shape=jax.ShapeDtypeStruct((1, 1), jnp.float32),
        scratch_shapes=[
            pltpu.VMEM((B, ROWS, LANE), jnp.float32),
            pltpu.VMEM((B, 1), jnp.float32),
            pltpu.VMEM((B, 1), jnp.float32),
            pltpu.VMEM((B, 1), jnp.float32),
        ],
    )(scores_t, locs_t, pri, boxes, bcxcy, labels32)
    return out[0, 0]


# speedup vs baseline: 1.0933x; 1.0027x over previous
import jax
import jax.numpy as jnp
from jax import lax
from jax.experimental import pallas as pl
from jax.experimental.pallas import tpu as pltpu

B = 32
P = 8732
C = 21
M = 20
ROWS = 69
LANE = 128
PP = ROWS * LANE
FULL = P // LANE
TAIL = P - FULL * LANE
THRESHOLD = 0.5
NEG_POS_RATIO = 3
ALPHA = 1.0
BIG = 2 ** 30


def _body(scores_ref, locs_ref, priors_ref, bxy_ref, bcxcy_ref, labels_ref,
          out_ref, ce_ref, npos_ref, confpos_ref, locsum_ref):
    i = pl.program_id(0)

    r_iota = lax.broadcasted_iota(jnp.int32, (ROWS, LANE), 0)
    l_iota = lax.broadcasted_iota(jnp.int32, (ROWS, LANE), 1)
    p_idx = r_iota * LANE + l_iota
    valid = p_idx < P

    px0 = priors_ref[0]
    py0 = priors_ref[1]
    px1 = priors_ref[2]
    py1 = priors_ref[3]
    pcx = priors_ref[4]
    pcy = priors_ref[5]
    pw = priors_ref[6]
    ph = priors_ref[7]
    parea = (px1 - px0) * (py1 - py0)

    rows = []
    for m in range(M):
        bx0 = bxy_ref[0, m, 0]
        by0 = bxy_ref[0, m, 1]
        bx1 = bxy_ref[0, m, 2]
        by1 = bxy_ref[0, m, 3]
        iw = jnp.clip(jnp.minimum(bx1, px1) - jnp.maximum(bx0, px0), 0.0)
        ih = jnp.clip(jnp.minimum(by1, py1) - jnp.maximum(by0, py0), 0.0)
        inter = iw * ih
        barea = (bx1 - bx0) * (by1 - by0)
        rows.append(inter / (barea + parea - inter + 1e-8))
    iou3 = jnp.stack(rows, axis=0)
    m_iota = lax.broadcasted_iota(jnp.int32, (M, ROWS, LANE), 0)
    p_idx3 = p_idx[None]

    best_val = jnp.max(iou3, axis=0)
    best_obj = jnp.min(jnp.where(iou3 == best_val[None], m_iota, BIG),
                       axis=0)
    row_max = jnp.max(iou3, axis=(1, 2), keepdims=True)
    prior_m = jnp.min(jnp.where(iou3 == row_max, p_idx3, BIG),
                      axis=(1, 2), keepdims=True)

    eqp = p_idx3 == prior_m
    forced = jnp.max(jnp.where(eqp, m_iota, -1), axis=0)
    is_forced = forced >= 0
    best_obj = jnp.where(is_forced, forced, best_obj)
    best_val = jnp.where(is_forced, 1.0, best_val)

    lab = jnp.zeros((ROWS, LANE), jnp.int32)
    bcx = jnp.zeros((ROWS, LANE), jnp.float32)
    bcy = jnp.zeros((ROWS, LANE), jnp.float32)
    bw = jnp.zeros((ROWS, LANE), jnp.float32)
    bh = jnp.zeros((ROWS, LANE), jnp.float32)
    for m in range(M):
        is_m = best_obj == m
        lab = jnp.where(is_m, labels_ref[0, 0, m], lab)
        bcx = jnp.where(is_m, bcxcy_ref[0, m, 0], bcx)
        bcy = jnp.where(is_m, bcxcy_ref[0, m, 1], bcy)
        bw = jnp.where(is_m, bcxcy_ref[0, m, 2], bw)
        bh = jnp.where(is_m, bcxcy_ref[0, m, 3], bh)
    lab = jnp.where(best_val < THRESHOLD, 0, lab)
    positive = lab != 0
    n_pos = jnp.sum(positive.astype(jnp.float32), keepdims=True)

    g0 = (bcx - pcx) / (pw / 10.0)
    g1 = (bcy - pcy) / (ph / 10.0)
    g2 = jnp.log(jnp.maximum(bw / pw, 1e-8)) * 5.0
    g3 = jnp.log(jnp.maximum(bh / ph, 1e-8)) * 5.0
    sl1_sum = jnp.zeros((ROWS, LANE), jnp.float32)
    for j, g in enumerate((g0, g1, g2, g3)):
        d = locs_ref[0, j] - g
        ad = jnp.abs(d)
        sl1_sum = sl1_sum + jnp.where(ad < 1.0, 0.5 * ad * ad, ad - 0.5)
    loc_sum = jnp.sum(jnp.where(positive, sl1_sum, 0.0), keepdims=True)

    maxv = scores_ref[0, 0]
    for c in range(1, C):
        maxv = jnp.maximum(maxv, scores_ref[0, c])
    sumexp = jnp.zeros((ROWS, LANE), jnp.float32)
    s_lab = jnp.zeros((ROWS, LANE), jnp.float32)
    for c in range(C):
        s = scores_ref[0, c]
        sumexp = sumexp + jnp.exp(s - maxv)
        s_lab = jnp.where(lab == c, s, s_lab)
    ce = (maxv - s_lab) + jnp.log(sumexp)
    conf_pos = jnp.sum(jnp.where(positive, ce, 0.0), keepdims=True)
    ce_neg = jnp.where(positive | jnp.logical_not(valid), 0.0, ce)

    ce_ref[i] = ce_neg
    npos_ref[pl.ds(i, 1), :] = n_pos
    confpos_ref[pl.ds(i, 1), :] = conf_pos
    locsum_ref[pl.ds(i, 1), :] = loc_sum

    @pl.when(i == B - 1)
    def finalize():
        x = ce_ref[...]
        xb = lax.bitcast_convert_type(x, jnp.int32)
        k_f = npos_ref[...].reshape(B, 1, 1) * float(NEG_POS_RATIO)

        def search_body(t, u):
            v = u | (jnp.int32(1) << (jnp.int32(30) - t))
            cnt = jnp.sum(jnp.where(xb >= v, 1.0, 0.0),
                          axis=(1, 2), keepdims=True)
            return jnp.where(cnt >= k_f, v, u)

        u = lax.fori_loop(0, 31, search_body,
                          jnp.zeros((B, 1, 1), jnp.int32))
        gt = xb > u
        cnt_gt = jnp.sum(jnp.where(gt, 1.0, 0.0), axis=(1, 2), keepdims=True)
        sum_gt = jnp.sum(jnp.where(gt, x, 0.0), axis=(1, 2), keepdims=True)
        t_star = jnp.max(jnp.where(xb <= u, x, 0.0),
                         axis=(1, 2), keepdims=True)
        topk = sum_gt + (k_f - cnt_gt) * t_star

        conf_total = (jnp.sum(confpos_ref[...], axis=0, keepdims=True)
                      + jnp.sum(topk, axis=0))
        loc_total = jnp.sum(locsum_ref[...], axis=0, keepdims=True)
        n_total = jnp.maximum(jnp.sum(npos_ref[...], axis=0, keepdims=True),
                              1.0)
        out_ref[:, :] = conf_total / n_total + ALPHA * loc_total / (n_total * 4.0)


def kernel(predicted_locs, predicted_scores, boxes, labels, priors_cxcy):
    pad = PP - P
    scores_t = jnp.transpose(
        jnp.pad(predicted_scores, ((0, 0), (0, pad), (0, 0))),
        (0, 2, 1)).reshape(B, C, ROWS, LANE)
    locs_t = jnp.transpose(
        jnp.pad(predicted_locs, ((0, 0), (0, pad), (0, 0))),
        (0, 2, 1)).reshape(B, 4, ROWS, LANE)
    priors_xy = jnp.concatenate(
        [priors_cxcy[..., :2] - priors_cxcy[..., 2:] / 2.0,
         priors_cxcy[..., :2] + priors_cxcy[..., 2:] / 2.0], axis=-1)
    pri = jnp.pad(jnp.transpose(
        jnp.concatenate([priors_xy, priors_cxcy], axis=-1), (1, 0)),
        ((0, 0), (0, pad))).reshape(8, ROWS, LANE)
    bcxcy = jnp.concatenate([(boxes[..., :2] + boxes[..., 2:]) / 2.0,
                             boxes[..., 2:] - boxes[..., :2]], axis=-1)
    labels32 = labels.astype(jnp.int32).reshape(B, 1, M)

    grid = (B,)

    def img_map(i):
        return (i, 0, 0, 0)

    def img_map3(i):
        return (i, 0, 0)

    out = pl.pallas_call(
        _body,
        grid=grid,
        in_specs=[
            pl.BlockSpec((1, C, ROWS, LANE), img_map),
            pl.BlockSpec((1, 4, ROWS, LANE), img_map),
            pl.BlockSpec((8, ROWS, LANE), lambda i: (0, 0, 0)),
            pl.BlockSpec((1, M, 4), img_map3, memory_space=pltpu.SMEM),
            pl.BlockSpec((1, M, 4), img_map3, memory_space=pltpu.SMEM),
            pl.BlockSpec((1, 1, M), img_map3, memory_space=pltpu.SMEM),
        ],
        out_specs=pl.BlockSpec((1, 1), lambda i: (0, 0)),
        out_shape=jax.ShapeDtypeStruct((1, 1), jnp.float32),
        scratch_shapes=[
            pltpu.VMEM((B, ROWS, LANE), jnp.float32),
            pltpu.VMEM((B, 1), jnp.float32),
            pltpu.VMEM((B, 1), jnp.float32),
            pltpu.VMEM((B, 1), jnp.float32),
        ],
    )(scores_t, locs_t, pri, boxes, bcxcy, labels32)
    return out[0, 0]


# speedup vs baseline: 1.0952x; 1.0017x over previous
import jax
import jax.numpy as jnp
from jax import lax
from jax.experimental import pallas as pl
from jax.experimental.pallas import tpu as pltpu

B = 32
P = 8732
C = 21
M = 20
ROWS = 69
LANE = 128
PP = ROWS * LANE
THRESHOLD = 0.5
NEG_POS_RATIO = 3
ALPHA = 1.0
BIG = 2 ** 30


def _body(scores_ref, locs_ref, priors_ref, bxy_ref, bcxcy_ref, labels_ref,
          out_ref, ce_ref, npos_ref, confpos_ref, locsum_ref):
    i = pl.program_id(0)

    r_iota = lax.broadcasted_iota(jnp.int32, (ROWS, LANE), 0)
    l_iota = lax.broadcasted_iota(jnp.int32, (ROWS, LANE), 1)
    p_idx = r_iota * LANE + l_iota
    valid = p_idx < P

    px0 = priors_ref[0]
    py0 = priors_ref[1]
    px1 = priors_ref[2]
    py1 = priors_ref[3]
    pcx = priors_ref[4]
    pcy = priors_ref[5]
    pw = priors_ref[6]
    ph = priors_ref[7]
    parea = (px1 - px0) * (py1 - py0)

    rows = []
    for m in range(M):
        bx0 = bxy_ref[0, m, 0]
        by0 = bxy_ref[0, m, 1]
        bx1 = bxy_ref[0, m, 2]
        by1 = bxy_ref[0, m, 3]
        iw = jnp.clip(jnp.minimum(bx1, px1) - jnp.maximum(bx0, px0), 0.0)
        ih = jnp.clip(jnp.minimum(by1, py1) - jnp.maximum(by0, py0), 0.0)
        inter = iw * ih
        barea = (bx1 - bx0) * (by1 - by0)
        rows.append(inter / (barea + parea - inter + 1e-8))
    iou3 = jnp.stack(rows, axis=0)
    m_iota = lax.broadcasted_iota(jnp.int32, (M, ROWS, LANE), 0)
    p_idx3 = p_idx[None]

    best_val = jnp.max(iou3, axis=0)
    best_obj = jnp.min(jnp.where(iou3 == best_val[None], m_iota, BIG),
                       axis=0)
    row_max = jnp.max(iou3, axis=(1, 2), keepdims=True)
    prior_m = jnp.min(jnp.where(iou3 == row_max, p_idx3, BIG),
                      axis=(1, 2), keepdims=True)

    eqp = p_idx3 == prior_m
    forced = jnp.max(jnp.where(eqp, m_iota, -1), axis=0)
    is_forced = forced >= 0
    best_obj = jnp.where(is_forced, forced, best_obj)
    best_val = jnp.where(is_forced, 1.0, best_val)

    lab = jnp.zeros((ROWS, LANE), jnp.int32)
    bcx = jnp.zeros((ROWS, LANE), jnp.float32)
    bcy = jnp.zeros((ROWS, LANE), jnp.float32)
    bw = jnp.zeros((ROWS, LANE), jnp.float32)
    bh = jnp.zeros((ROWS, LANE), jnp.float32)
    for m in range(M):
        is_m = best_obj == m
        lab = jnp.where(is_m, labels_ref[0, 0, m], lab)
        bcx = jnp.where(is_m, bcxcy_ref[0, m, 0], bcx)
        bcy = jnp.where(is_m, bcxcy_ref[0, m, 1], bcy)
        bw = jnp.where(is_m, bcxcy_ref[0, m, 2], bw)
        bh = jnp.where(is_m, bcxcy_ref[0, m, 3], bh)
    lab = jnp.where(best_val < THRESHOLD, 0, lab)
    positive = lab != 0
    n_pos = jnp.sum(positive.astype(jnp.float32), keepdims=True)

    g0 = (bcx - pcx) / (pw / 10.0)
    g1 = (bcy - pcy) / (ph / 10.0)
    g2 = jnp.log(jnp.maximum(bw / pw, 1e-8)) * 5.0
    g3 = jnp.log(jnp.maximum(bh / ph, 1e-8)) * 5.0
    sl1_sum = jnp.zeros((ROWS, LANE), jnp.float32)
    for j, g in enumerate((g0, g1, g2, g3)):
        d = locs_ref[0, j] - g
        ad = jnp.abs(d)
        sl1_sum = sl1_sum + jnp.where(ad < 1.0, 0.5 * ad * ad, ad - 0.5)
    loc_sum = jnp.sum(jnp.where(positive, sl1_sum, 0.0), keepdims=True)

    maxv = scores_ref[0, 0]
    for c in range(1, C):
        maxv = jnp.maximum(maxv, scores_ref[0, c])
    sumexp = jnp.zeros((ROWS, LANE), jnp.float32)
    s_lab = jnp.zeros((ROWS, LANE), jnp.float32)
    for c in range(C):
        s = scores_ref[0, c]
        sumexp = sumexp + jnp.exp(s - maxv)
        s_lab = jnp.where(lab == c, s, s_lab)
    ce = (maxv - s_lab) + jnp.log(sumexp)
    conf_pos = jnp.sum(jnp.where(positive, ce, 0.0), keepdims=True)
    ce_neg = jnp.where(positive | jnp.logical_not(valid), 0.0, ce)

    ce_ref[i] = ce_neg
    npos_ref[pl.ds(i, 1), :] = n_pos
    confpos_ref[pl.ds(i, 1), :] = conf_pos
    locsum_ref[pl.ds(i, 1), :] = loc_sum

    @pl.when(i == B - 1)
    def finalize():
        x = ce_ref[...]
        xb = lax.bitcast_convert_type(x, jnp.int32)
        k_f = npos_ref[...].reshape(B, 1, 1) * float(NEG_POS_RATIO)

        def search_body(t, u):
            v = u | (jnp.int32(1) << (jnp.int32(30) - t))
            cnt = jnp.sum(jnp.where(xb >= v, 1.0, 0.0),
                          axis=(1, 2), keepdims=True)
            return jnp.where(cnt >= k_f, v, u)

        u = lax.fori_loop(0, 31, search_body,
                          jnp.zeros((B, 1, 1), jnp.int32))
        gt = xb > u
        cnt_gt = jnp.sum(jnp.where(gt, 1.0, 0.0), axis=(1, 2), keepdims=True)
        sum_gt = jnp.sum(jnp.where(gt, x, 0.0), axis=(1, 2), keepdims=True)
        t_star = jnp.max(jnp.where(xb <= u, x, 0.0),
                         axis=(1, 2), keepdims=True)
        topk = sum_gt + (k_f - cnt_gt) * t_star

        conf_total = (jnp.sum(confpos_ref[...], axis=0, keepdims=True)
                      + jnp.sum(topk, axis=0))
        loc_total = jnp.sum(locsum_ref[...], axis=0, keepdims=True)
        n_total = jnp.maximum(jnp.sum(npos_ref[...], axis=0, keepdims=True),
                              1.0)
        out_ref[:, :] = conf_total / n_total + ALPHA * loc_total / (n_total * 4.0)


def kernel(predicted_locs, predicted_scores, boxes, labels, priors_cxcy):
    pad = PP - P
    scores_t = jnp.transpose(
        jnp.pad(predicted_scores, ((0, 0), (0, pad), (0, 0))),
        (0, 2, 1)).reshape(B, C, ROWS, LANE)
    locs_t = jnp.transpose(
        jnp.pad(predicted_locs, ((0, 0), (0, pad), (0, 0))),
        (0, 2, 1)).reshape(B, 4, ROWS, LANE)
    priors_xy = jnp.concatenate(
        [priors_cxcy[..., :2] - priors_cxcy[..., 2:] / 2.0,
         priors_cxcy[..., :2] + priors_cxcy[..., 2:] / 2.0], axis=-1)
    pri = jnp.pad(jnp.transpose(
        jnp.concatenate([priors_xy, priors_cxcy], axis=-1), (1, 0)),
        ((0, 0), (0, pad))).reshape(8, ROWS, LANE)
    bcxcy = jnp.concatenate([(boxes[..., :2] + boxes[..., 2:]) / 2.0,
                             boxes[..., 2:] - boxes[..., :2]], axis=-1)
    labels32 = labels.astype(jnp.int32).reshape(B, 1, M)

    grid = (B,)

    def img_map(i):
        return (i, 0, 0, 0)

    def img_map3(i):
        return (i, 0, 0)

    out = pl.pallas_call(
        _body,
        grid=grid,
        in_specs=[
            pl.BlockSpec((1, C, ROWS, LANE), img_map),
            pl.BlockSpec((1, 4, ROWS, LANE), img_map),
            pl.BlockSpec((8, ROWS, LANE), lambda i: (0, 0, 0)),
            pl.BlockSpec((1, M, 4), img_map3, memory_space=pltpu.SMEM),
            pl.BlockSpec((1, M, 4), img_map3, memory_space=pltpu.SMEM),
            pl.BlockSpec((1, 1, M), img_map3, memory_space=pltpu.SMEM),
        ],
        out_specs=pl.BlockSpec((1, 1), lambda i: (0, 0)),
        out_shape=jax.ShapeDtypeStruct((1, 1), jnp.float32),
        scratch_shapes=[
            pltpu.VMEM((B, ROWS, LANE), jnp.float32),
            pltpu.VMEM((B, 1), jnp.float32),
            pltpu.VMEM((B, 1), jnp.float32),
            pltpu.VMEM((B, 1), jnp.float32),
        ],
    )(scores_t, locs_t, pri, boxes, bcxcy, labels32)
    return out[0, 0]
